# baseline (speedup 1.0000x reference)
import functools

import jax
import jax.numpy as jnp
from jax.experimental import pallas as pl
from jax.experimental.pallas import tpu as pltpu

_B = 2
_K = 8
_L = 4


def _se_ring_kernel(x_hbm, w1_ref, b1_ref, w2_ref, b2_ref, o_hbm,
                    buf, sem_in, sem_out, *, chunks_per_core, B, K, L, inv_hw):
    core = pl.program_id(0)
    ncores = pl.num_programs(0)

    def in_copy(s, k):
        idx = (s * ncores + core) * B
        return pltpu.make_async_copy(
            x_hbm.at[pl.ds(idx, B)], buf.at[k], sem_in.at[k])

    def out_copy(s, k):
        idx = (s * ncores + core) * B
        return pltpu.make_async_copy(
            buf.at[k], o_hbm.at[pl.ds(idx, B)], sem_out.at[k])

    for s in range(L):
        in_copy(s, s % K).start()

    for s in range(chunks_per_core):
        ns = s + L
        if ns < chunks_per_core:
            k2 = ns % K
            if ns >= K:
                out_copy(ns - K, k2).wait()
            in_copy(ns, k2).start()

        k = s % K
        in_copy(s, k).wait()

        x = buf[k]
        pooled = jnp.sum(x, axis=2) * inv_hw
        h = jnp.dot(pooled, w1_ref[...],
                    preferred_element_type=jnp.float32) + b1_ref[...]
        h = jnp.maximum(h, 0.0)
        y = jax.nn.sigmoid(
            jnp.dot(h, w2_ref[...],
                    preferred_element_type=jnp.float32) + b2_ref[...])
        buf[k] = x * y[:, :, None]

        out_copy(s, k).start()

    for s in range(max(0, chunks_per_core - K), chunks_per_core):
        out_copy(s, s % K).wait()


def kernel(x, w1, b1, w2, b2):
    N, C, H, W = x.shape
    Cr = w1.shape[1]
    HW = H * W

    x_flat = x.reshape(N, C, HW)
    b1r = b1.reshape(1, Cr)
    b2r = b2.reshape(1, C)

    cores = 2 if N % 2 == 0 else 1
    imgs_per_core = N // cores
    B = _B if imgs_per_core % _B == 0 else 1
    chunks_per_core = imgs_per_core // B
    K = min(_K, chunks_per_core)
    L = min(_L, K)

    out_flat = pl.pallas_call(
        functools.partial(_se_ring_kernel,
                          chunks_per_core=chunks_per_core,
                          B=B, K=K, L=L,
                          inv_hw=1.0 / float(HW)),
        out_shape=jax.ShapeDtypeStruct((N, C, HW), x.dtype),
        grid=(cores,),
        in_specs=[
            pl.BlockSpec(memory_space=pltpu.MemorySpace.HBM),
            pl.BlockSpec((C, Cr), lambda i: (0, 0)),
            pl.BlockSpec((1, Cr), lambda i: (0, 0)),
            pl.BlockSpec((Cr, C), lambda i: (0, 0)),
            pl.BlockSpec((1, C), lambda i: (0, 0)),
        ],
        out_specs=pl.BlockSpec(memory_space=pltpu.MemorySpace.HBM),
        scratch_shapes=[
            pltpu.VMEM((K, B, C, HW), jnp.float32),
            pltpu.SemaphoreType.DMA((K,)),
            pltpu.SemaphoreType.DMA((K,)),
        ],
        compiler_params=pltpu.CompilerParams(
            dimension_semantics=("parallel",),
            vmem_limit_bytes=64 << 20,
        ),
    )(x_flat, w1, b1r, w2, b2r)

    return out_flat.reshape(N, C, H, W)


# speedup vs baseline: 1.0039x; 1.0039x over previous
import functools

import jax
import jax.numpy as jnp
from jax.experimental import pallas as pl
from jax.experimental.pallas import tpu as pltpu

_NB = 4


def _se_hybrid_kernel(x_ref, w1_ref, b1_ref, w2_ref, b2_ref, o_hbm,
                      obuf, sem, *, Nb, steps_per_core, inv_hw):
    c = pl.program_id(0)
    j = pl.program_id(1)
    slot = jax.lax.rem(j, 2)
    idx0 = (c * steps_per_core + j) * Nb

    def out_copy(b, image_idx, slot_):
        return pltpu.make_async_copy(
            obuf.at[slot_, pl.ds(b, 1)],
            o_hbm.at[pl.ds(image_idx, 1)],
            sem.at[slot_, b])

    @pl.when(j >= 2)
    def _():
        for b in range(Nb):
            out_copy(b, idx0 + b, slot).wait()

    x = x_ref[...]
    pooled = jnp.sum(x, axis=2) * inv_hw
    h = jnp.dot(pooled, w1_ref[...],
                preferred_element_type=jnp.float32) + b1_ref[...]
    h = jnp.maximum(h, 0.0)
    y = jax.nn.sigmoid(
        jnp.dot(h, w2_ref[...],
                preferred_element_type=jnp.float32) + b2_ref[...])

    for b in range(Nb):
        obuf[slot, b] = x[b] * y[b, :, None]
        out_copy(b, idx0 + b, slot).start()

    @pl.when(j == steps_per_core - 1)
    def _():
        for b in range(Nb):
            out_copy(b, idx0 + b, slot).wait()
        if steps_per_core > 1:
            for b in range(Nb):
                out_copy(b, idx0 + b, 1 - slot).wait()


def kernel(x, w1, b1, w2, b2):
    N, C, H, W = x.shape
    Cr = w1.shape[1]
    HW = H * W

    x_flat = x.reshape(N, C, HW)
    b1r = b1.reshape(1, Cr)
    b2r = b2.reshape(1, C)

    Nb = _NB if N % (2 * _NB) == 0 else 1
    cores = 2 if N % 2 == 0 else 1
    steps_per_core = N // (cores * Nb)

    out_flat = pl.pallas_call(
        functools.partial(_se_hybrid_kernel,
                          Nb=Nb, steps_per_core=steps_per_core,
                          inv_hw=1.0 / float(HW)),
        out_shape=jax.ShapeDtypeStruct((N, C, HW), x.dtype),
        grid=(cores, steps_per_core),
        in_specs=[
            pl.BlockSpec((Nb, C, HW),
                         lambda c, j, spc=steps_per_core: (c * spc + j, 0, 0)),
            pl.BlockSpec((C, Cr), lambda c, j: (0, 0)),
            pl.BlockSpec((1, Cr), lambda c, j: (0, 0)),
            pl.BlockSpec((Cr, C), lambda c, j: (0, 0)),
            pl.BlockSpec((1, C), lambda c, j: (0, 0)),
        ],
        out_specs=pl.BlockSpec(memory_space=pltpu.MemorySpace.HBM),
        scratch_shapes=[
            pltpu.VMEM((2, Nb, C, HW), jnp.float32),
            pltpu.SemaphoreType.DMA((2, Nb)),
        ],
        compiler_params=pltpu.CompilerParams(
            dimension_semantics=("parallel", "arbitrary"),
            vmem_limit_bytes=64 << 20,
        ),
    )(x_flat, w1, b1r, w2, b2r)

    return out_flat.reshape(N, C, H, W)
